# baseline (speedup 1.0000x reference)
import functools

import jax
import jax.numpy as jnp
from jax import lax
from jax.experimental import pallas as pl
from jax.experimental.pallas import tpu as pltpu
from jax.experimental.pallas import tpu_sc as plsc

NC = 2
NS = 16
NW = NC * NS
R = 4
SPLITS = ((0, 64), (64, 64), (128, 72))


@functools.cache
def _make_gather(B: int, L: int, V: int, D: int):
    assert B % (NW * R) == 0
    assert sum(g for _, g in SPLITS) == L
    b_per_w = B // NW
    n_s = b_per_w // R
    assert n_s % 2 == 0
    mesh = plsc.VectorSubcoreMesh(core_axis_name="c", subcore_axis_name="s")

    @functools.partial(
        pl.kernel,
        mesh=mesh,
        out_type=jax.ShapeDtypeStruct((B, L, 2 * D), jnp.float32),
        compiler_params=pltpu.CompilerParams(use_tc_tiling_on_sc=False),
        scratch_types=[
            pltpu.VMEM((b_per_w, L), jnp.int32),
            pltpu.VMEM((2, R, L, D), jnp.float32),
            pltpu.SemaphoreType.DMA,
            pltpu.SemaphoreType.DMA,
            pltpu.SemaphoreType.DMA,
            pltpu.SemaphoreType.DMA,
        ],
    )
    def gather_kernel(table_hbm, seqs_hbm, out_hbm, idx_v, slab_v,
                      gsem0, gsem1, osem0, osem1):
        wid = lax.axis_index("s") * NC + lax.axis_index("c")
        b0 = wid * b_per_w

        pltpu.sync_copy(seqs_hbm.at[pl.ds(b0, b_per_w)], idx_v)

        gsems = (gsem0, gsem1)
        osems = (osem0, osem1)

        def fill(s, p):
            for r in range(R):
                for o, g in SPLITS:
                    pltpu.async_copy(
                        table_hbm.at[idx_v.at[s * R + r, pl.ds(o, g)]],
                        slab_v.at[p, r, pl.ds(o, g)],
                        gsems[p],
                    )

        def drain(s, p):
            for r in range(R):
                for o, g in SPLITS:
                    pltpu.make_async_copy(
                        table_hbm.at[idx_v.at[s * R + r, pl.ds(o, g)]],
                        slab_v.at[p, r, pl.ds(o, g)],
                        gsems[p],
                    ).wait()

        def out_slice(s):
            return out_hbm.at[pl.ds(b0 + s * R, R), :, pl.ds(0, D)]

        fill(0, 0)

        def body(t, _):
            for p in range(2):
                s = t * 2 + p
                q = 1 - p

                @pl.when(s + 1 < n_s)
                def _():
                    @pl.when(s >= 1)
                    def _():
                        pltpu.make_async_copy(
                            slab_v.at[q], out_slice(s - 1), osems[q]
                        ).wait()
                    fill(s + 1, q)

                drain(s, p)
                pltpu.async_copy(slab_v.at[p], out_slice(s), osems[p])
            return 0

        lax.fori_loop(0, n_s // 2, body, 0)

        pltpu.make_async_copy(slab_v.at[0], out_slice(n_s - 2), osems[0]).wait()
        pltpu.make_async_copy(slab_v.at[1], out_slice(n_s - 1), osems[1]).wait()

    return gather_kernel


def kernel(seqs, att_mask, word_embedding):
    B, L = seqs.shape
    V, D = word_embedding.shape
    padded = _make_gather(B, L, V, D)(word_embedding, seqs.astype(jnp.int32))
    return padded[:, :, :D]


# speedup vs baseline: 1.0014x; 1.0014x over previous
import functools

import jax
import jax.numpy as jnp
from jax import lax
from jax.experimental import pallas as pl
from jax.experimental.pallas import tpu as pltpu
from jax.experimental.pallas import tpu_sc as plsc

NC = 2
NS = 16
NW = NC * NS
R = 2
NB = 4
SPLITS = ((0, 128), (128, 72))


@functools.cache
def _make_gather(B: int, L: int, V: int, D: int):
    assert B % (NW * R) == 0
    assert sum(g for _, g in SPLITS) == L
    b_per_w = B // NW
    n_s = b_per_w // R
    assert n_s % NB == 0 and n_s > 2 * NB
    mesh = plsc.VectorSubcoreMesh(core_axis_name="c", subcore_axis_name="s")

    @functools.partial(
        pl.kernel,
        mesh=mesh,
        out_type=jax.ShapeDtypeStruct((B, L, 2 * D), jnp.float32),
        compiler_params=pltpu.CompilerParams(use_tc_tiling_on_sc=False),
        scratch_types=[
            pltpu.VMEM((b_per_w, L), jnp.int32),
            pltpu.VMEM((NB, R, L, D), jnp.float32),
        ] + [pltpu.SemaphoreType.DMA] * (2 * NB),
    )
    def gather_kernel(table_hbm, seqs_hbm, out_hbm, idx_v, slab_v, *sems):
        wid = lax.axis_index("s") * NC + lax.axis_index("c")
        b0 = wid * b_per_w

        pltpu.sync_copy(seqs_hbm.at[pl.ds(b0, b_per_w)], idx_v)

        gsems = sems[:NB]
        osems = sems[NB:]

        def fill(s, p):
            for r in range(R):
                for o, g in SPLITS:
                    pltpu.async_copy(
                        table_hbm.at[idx_v.at[s * R + r, pl.ds(o, g)]],
                        slab_v.at[p, r, pl.ds(o, g)],
                        gsems[p],
                    )

        def drain(s, p):
            for r in range(R):
                for o, g in SPLITS:
                    pltpu.make_async_copy(
                        table_hbm.at[idx_v.at[s * R + r, pl.ds(o, g)]],
                        slab_v.at[p, r, pl.ds(o, g)],
                        gsems[p],
                    ).wait()

        def out_slice(s):
            return out_hbm.at[pl.ds(b0 + s * R, R), :, pl.ds(0, D)]

        for p in range(NB - 1):
            fill(p, p)

        def body(t, _):
            for p0 in range(NB):
                s = t * NB + p0
                q = (p0 + NB - 1) % NB

                @pl.when(s + NB - 1 < n_s)
                def _():
                    @pl.when(s >= 1)
                    def _():
                        pltpu.make_async_copy(
                            slab_v.at[q], out_slice(s - 1), osems[q]
                        ).wait()
                    fill(s + NB - 1, q)

                drain(s, p0)
                pltpu.async_copy(slab_v.at[p0], out_slice(s), osems[p0])
            return 0

        lax.fori_loop(0, n_s // NB, body, 0)

        for k in range(NB):
            s = n_s - NB + k
            pltpu.make_async_copy(
                slab_v.at[s % NB], out_slice(s), osems[s % NB]).wait()

    return gather_kernel


def kernel(seqs, att_mask, word_embedding):
    B, L = seqs.shape
    V, D = word_embedding.shape
    padded = _make_gather(B, L, V, D)(word_embedding, seqs.astype(jnp.int32))
    return padded[:, :, :D]
